# baseline (speedup 1.0000x reference)
import functools

import jax
import jax.numpy as jnp
from jax import lax
from jax.experimental import pallas as pl
from jax.experimental.pallas import tpu as pltpu
from jax.experimental.pallas import tpu_sc as plsc

INPUT_DIM = 100000
OUTPUT_DIM = 64
INPUT_LENGTH = 200
BATCH = 4096

_NC = 2
_NS = 16
_NW = _NC * _NS
_NBUF = 4


def _make_body(b_per_w):
    def _emb_body(x_hbm, tok_hbm, pos_hbm, out_hbm, idx_v, pos_v,
                  r0, r1, r2, r3, g0, g1, g2, g3, o0, o1, o2, o3):
        bufs = (r0, r1, r2, r3)
        gsems = (g0, g1, g2, g3)
        osems = (o0, o1, o2, o3)
        wid = lax.axis_index("s") * _NC + lax.axis_index("c")
        base = wid * b_per_w
        pltpu.sync_copy(pos_hbm, pos_v)
        pltpu.sync_copy(x_hbm.at[pl.ds(base, b_per_w)], idx_v)

        def gather(b, p):
            return pltpu.make_async_copy(
                tok_hbm.at[idx_v.at[b]], bufs[p], gsems[p])

        def out_copy(b, p):
            return pltpu.make_async_copy(
                bufs[p], out_hbm.at[base + b], osems[p])

        def add_pos(p):
            buf = bufs[p]

            @plsc.parallel_loop(0, INPUT_LENGTH, unroll=8)
            def _(r):
                for c in range(OUTPUT_DIM // 16):
                    sl = pl.ds(c * 16, 16)
                    plsc.addupdate(buf.at[r, sl], pos_v[r, sl])

        for bb in range(2):
            gather(bb, bb).start()

        def gbody(g, carry):
            for p in range(_NBUF):
                b = _NBUF * g + p
                gather(b, p).wait()
                add_pos(p)
                out_copy(b, p).start()
                pf = (p + 2) % _NBUF
                if p < 2:
                    @pl.when(g > 0)
                    def _():
                        out_copy(b - 2, pf).wait()
                    gather(b + 2, pf).start()
                else:
                    out_copy(b - 2, pf).wait()

                    @pl.when(g < b_per_w // _NBUF - 1)
                    def _():
                        gather(b + 2, pf).start()
            return carry

        lax.fori_loop(0, b_per_w // _NBUF, gbody, 0)
        out_copy(b_per_w - 2, 2).wait()
        out_copy(b_per_w - 1, 3).wait()

    return _emb_body


@functools.partial(jax.jit, static_argnums=(3,))
def _emb_call(x3, token_table, pos_table, nbatch):
    b_per_w = nbatch // _NW
    mesh = plsc.VectorSubcoreMesh(core_axis_name="c", subcore_axis_name="s")
    run = pl.kernel(
        _make_body(b_per_w),
        mesh=mesh,
        out_type=jax.ShapeDtypeStruct((nbatch, INPUT_LENGTH, OUTPUT_DIM), jnp.float32),
        scratch_types=[
            pltpu.VMEM((b_per_w, INPUT_LENGTH), jnp.int32),
            pltpu.VMEM((INPUT_LENGTH, OUTPUT_DIM), jnp.float32),
        ] + [pltpu.VMEM((INPUT_LENGTH, OUTPUT_DIM), jnp.float32)] * _NBUF
          + [pltpu.SemaphoreType.DMA] * (2 * _NBUF),
        compiler_params=pltpu.CompilerParams(use_tc_tiling_on_sc=False),
    )
    return run(x3, token_table, pos_table)


_TC_B = 1024


def kernel(x, token_table, pos_table):
    x3 = x.astype(jnp.int32)
    sc_out = _emb_call(x3[_TC_B:], token_table, pos_table, BATCH - _TC_B)
    tc_out = jnp.take(token_table, x3[:_TC_B], axis=0) + pos_table[None, :, :]
    return jnp.concatenate([tc_out, sc_out], axis=0)


# speedup vs baseline: 2.1537x; 2.1537x over previous
import functools

import jax
import jax.numpy as jnp
from jax import lax
from jax.experimental import pallas as pl
from jax.experimental.pallas import tpu as pltpu
from jax.experimental.pallas import tpu_sc as plsc

INPUT_DIM = 100000
OUTPUT_DIM = 64
INPUT_LENGTH = 200
BATCH = 4096

_NC = 2
_NS = 16
_NW = _NC * _NS
_SEQ_PER_CHUNK = 2
_CHUNK = _SEQ_PER_CHUNK * INPUT_LENGTH
_C_PER_W = BATCH // _NW // _SEQ_PER_CHUNK
_NBUF = 4


def _emb_body(x_hbm, tok_hbm, pos_hbm, out_hbm,
              pos_v, i0, i1, i2, i3, r0, r1, r2, r3,
              g0, g1, g2, g3, o0, o1, o2, o3, s0, s1, s2, s3):
    bufs = (r0, r1, r2, r3)
    idxs = (i0, i1, i2, i3)
    gsems = (g0, g1, g2, g3)
    osems = (o0, o1, o2, o3)
    isems = (s0, s1, s2, s3)
    wid = lax.axis_index("s") * _NC + lax.axis_index("c")
    cbase = wid * _C_PER_W
    pltpu.sync_copy(pos_hbm, pos_v)

    def idx_copy(c, q):
        return pltpu.make_async_copy(
            x_hbm.at[pl.ds((cbase + c) * _CHUNK, _CHUNK)], idxs[q], isems[q])

    def gather(c, q, p):
        return pltpu.make_async_copy(
            tok_hbm.at[idxs[q]], bufs[p], gsems[p])

    def out_copy(c, p):
        return pltpu.make_async_copy(bufs[p], out_hbm.at[cbase + c], osems[p])

    def add_pos(p):
        buf = bufs[p]

        @plsc.parallel_loop(0, INPUT_LENGTH, unroll=8)
        def _(r):
            for c in range(OUTPUT_DIM // 16):
                sl = pl.ds(c * 16, 16)
                v = pos_v[r, sl]
                plsc.addupdate(buf.at[r, sl], v)
                plsc.addupdate(buf.at[r + INPUT_LENGTH, sl], v)

    for cc in range(3):
        idx_copy(cc, cc).start()
    for cc in range(2):
        idx_copy(cc, cc).wait()
        gather(cc, cc, cc).start()

    def gbody(g, carry):
        for p in range(_NBUF):
            c = _NBUF * g + p
            gather(c, p, p).wait()
            add_pos(p)
            out_copy(c, p).start()
            pf = (p + 2) % _NBUF
            if p < 2:
                @pl.when(g > 0)
                def _():
                    out_copy(c - 2, pf).wait()
                idx_copy(c + 2, pf).wait()
                gather(c + 2, pf, pf).start()
                if p == 0:
                    idx_copy(c + 3, (p + 3) % _NBUF).start()
                else:
                    @pl.when(g < _C_PER_W // _NBUF - 1)
                    def _():
                        idx_copy(c + 3, (p + 3) % _NBUF).start()
            else:
                out_copy(c - 2, pf).wait()
                last = g >= _C_PER_W // _NBUF - 1

                @pl.when(jnp.logical_not(last))
                def _():
                    idx_copy(c + 2, pf).wait()
                    gather(c + 2, pf, pf).start()
                    idx_copy(c + 3, (p + 3) % _NBUF).start()
        return carry

    lax.fori_loop(0, _C_PER_W // _NBUF, gbody, 0)
    out_copy(_C_PER_W - 2, 2).wait()
    out_copy(_C_PER_W - 1, 3).wait()


@jax.jit
def _emb_call(x1, token_table, pos_table):
    mesh = plsc.VectorSubcoreMesh(core_axis_name="c", subcore_axis_name="s")
    run = pl.kernel(
        _emb_body,
        mesh=mesh,
        out_type=jax.ShapeDtypeStruct(
            (BATCH // _SEQ_PER_CHUNK, _CHUNK, OUTPUT_DIM), jnp.float32),
        scratch_types=[
            pltpu.VMEM((INPUT_LENGTH, OUTPUT_DIM), jnp.float32),
        ] + [pltpu.VMEM((_CHUNK,), jnp.int32)] * _NBUF
          + [pltpu.VMEM((_CHUNK, OUTPUT_DIM), jnp.float32)] * _NBUF
          + [pltpu.SemaphoreType.DMA] * (3 * _NBUF),
        compiler_params=pltpu.CompilerParams(use_tc_tiling_on_sc=False),
    )
    return run(x1, token_table, pos_table)


def kernel(x, token_table, pos_table):
    x1 = x.astype(jnp.int32).reshape(-1)
    out = _emb_call(x1, token_table, pos_table)
    return out.reshape(BATCH, INPUT_LENGTH, OUTPUT_DIM)
